# baseline (speedup 1.0000x reference)
import functools

import jax
import jax.numpy as jnp
from jax import lax
from jax.experimental import pallas as pl
from jax.experimental.pallas import tpu as pltpu
from jax.experimental.pallas import tpu_sc as plsc

B = 16384
N_SPARSE = 26
N_DENSE = 7
N_COLS = N_SPARSE + N_DENSE
VOCAB = 1000

NUM_CORES = 2
NUM_SUBCORES = 16
NW = NUM_CORES * NUM_SUBCORES
ROWS_PER_W = B // NW
LANES = 16
GROUPS = ROWS_PER_W // LANES
TABLE_WORDS = N_SPARSE * VOCAB


@functools.partial(
    pl.kernel,
    mesh=plsc.VectorSubcoreMesh(core_axis_name="c", subcore_axis_name="s"),
    out_type=jax.ShapeDtypeStruct((B,), jnp.float32),
    compiler_params=pltpu.CompilerParams(needs_layout_passes=False),
    scratch_types=[
        pltpu.VMEM((N_COLS, ROWS_PER_W), jnp.float32),
        pltpu.VMEM((TABLE_WORDS,), jnp.float32),
        pltpu.VMEM((128,), jnp.float32),
        pltpu.VMEM((ROWS_PER_W,), jnp.float32),
        pltpu.SemaphoreType.DMA,
        pltpu.SemaphoreType.DMA,
    ],
)
def _linear_logit_sc(xt_hbm, t_hbm, w_hbm, out_hbm, xv, tv, wv, ov, semx, semt):
    wid = lax.axis_index("s") * NUM_CORES + lax.axis_index("c")
    base = wid * ROWS_PER_W
    xcp = pltpu.async_copy(xt_hbm.at[:, pl.ds(base, ROWS_PER_W)], xv, semx)
    pltpu.sync_copy(w_hbm.at[pl.ds(0, LANES)], wv.at[pl.ds(0, LANES)])
    tcp = pltpu.async_copy(t_hbm, tv, semt)
    wvec = wv[pl.ds(0, LANES)]
    wsplat = [wvec[d] for d in range(N_DENSE)]
    xcp.wait()

    @plsc.parallel_loop(0, GROUPS)
    def dense(g):
        r0 = g * LANES
        acc = xv[N_SPARSE, pl.ds(r0, LANES)] * wsplat[0]
        for d in range(1, N_DENSE):
            acc = acc + xv[N_SPARSE + d, pl.ds(r0, LANES)] * wsplat[d]
        ov[pl.ds(r0, LANES)] = acc

    tcp.wait()

    @plsc.parallel_loop(0, GROUPS)
    def sparse(g):
        r0 = g * LANES
        acc = ov[pl.ds(r0, LANES)]
        for f in range(N_SPARSE):
            ids = xv[f, pl.ds(r0, LANES)].astype(jnp.int32)
            acc = acc + plsc.load_gather(tv, [ids + f * VOCAB])
        ov[pl.ds(r0, LANES)] = acc

    pltpu.sync_copy(ov, out_hbm.at[pl.ds(base, ROWS_PER_W)])


def kernel(X, emb_tables, dense_weight):
    xt = X.T
    w_pad = jnp.pad(dense_weight.reshape(-1), (0, 128 - N_DENSE))
    out = _linear_logit_sc(xt, emb_tables.reshape(-1), w_pad)
    return out.reshape(B, 1)


# speedup vs baseline: 1.0354x; 1.0354x over previous
import functools

import jax
import jax.numpy as jnp
from jax import lax
from jax.experimental import pallas as pl
from jax.experimental.pallas import tpu as pltpu
from jax.experimental.pallas import tpu_sc as plsc

B = 16384
N_SPARSE = 26
N_DENSE = 7
N_COLS = N_SPARSE + N_DENSE
VOCAB = 1000

NUM_CORES = 2
NUM_SUBCORES = 16
NW = NUM_CORES * NUM_SUBCORES
ROWS_PER_W = B // NW
LANES = 16
GROUPS = ROWS_PER_W // LANES
TABLE_WORDS = N_SPARSE * VOCAB
FBLOCKS = (6, 6, 7, 7)


@functools.partial(
    pl.kernel,
    mesh=plsc.VectorSubcoreMesh(core_axis_name="c", subcore_axis_name="s"),
    out_type=jax.ShapeDtypeStruct((B,), jnp.float32),
    compiler_params=pltpu.CompilerParams(needs_layout_passes=False),
    scratch_types=[
        pltpu.VMEM((N_COLS, ROWS_PER_W), jnp.float32),
        pltpu.VMEM((TABLE_WORDS + 16,), jnp.float32),
        pltpu.VMEM((ROWS_PER_W,), jnp.float32),
        pltpu.SemaphoreType.DMA,
        pltpu.SemaphoreType.DMA,
        pltpu.SemaphoreType.DMA,
        pltpu.SemaphoreType.DMA,
        pltpu.SemaphoreType.DMA,
    ],
)
def _linear_logit_sc(
    xt_hbm, t_hbm, out_hbm, xv, tv, ov, semxd, st0, st1, st2, st3
):
    wid = lax.axis_index("s") * NUM_CORES + lax.axis_index("c")
    base = wid * ROWS_PER_W
    xcp = pltpu.async_copy(
        xt_hbm.at[:, pl.ds(base, ROWS_PER_W)], xv, semxd
    )
    pltpu.sync_copy(
        t_hbm.at[pl.ds(TABLE_WORDS, 8)], tv.at[pl.ds(TABLE_WORDS, 8)]
    )
    tsems = [st0, st1, st2, st3]
    tcps = []
    f0 = 0
    fstarts = []
    for bi, nf in enumerate(FBLOCKS):
        fstarts.append(f0)
        tcps.append(
            pltpu.async_copy(
                t_hbm.at[pl.ds(f0 * VOCAB, nf * VOCAB)],
                tv.at[pl.ds(f0 * VOCAB, nf * VOCAB)],
                tsems[bi],
            )
        )
        f0 += nf

    wvec = tv[pl.ds(TABLE_WORDS, LANES)]
    wsplat = [wvec[d] for d in range(N_DENSE)]
    xcp.wait()

    @plsc.parallel_loop(0, GROUPS)
    def dense(g):
        r0 = g * LANES
        acc = xv[N_SPARSE, pl.ds(r0, LANES)] * wsplat[0]
        for d in range(1, N_DENSE):
            acc = acc + xv[N_SPARSE + d, pl.ds(r0, LANES)] * wsplat[d]
        ov[pl.ds(r0, LANES)] = acc

    for bi, nf in enumerate(FBLOCKS):
        tcps[bi].wait()
        fs = fstarts[bi]

        @plsc.parallel_loop(0, GROUPS)
        def sparse(g, fs=fs, nf=nf):
            r0 = g * LANES
            acc = ov[pl.ds(r0, LANES)]
            for f in range(fs, fs + nf):
                ids = xv[f, pl.ds(r0, LANES)].astype(jnp.int32)
                acc = acc + plsc.load_gather(tv, [ids + f * VOCAB])
            ov[pl.ds(r0, LANES)] = acc

    pltpu.sync_copy(ov, out_hbm.at[pl.ds(base, ROWS_PER_W)])


def kernel(X, emb_tables, dense_weight):
    xt = X.T
    t_flat = jnp.concatenate([
        emb_tables.reshape(-1),
        jnp.pad(dense_weight.reshape(-1), (0, 8 - N_DENSE)),
    ])
    out = _linear_logit_sc(xt, t_flat)
    return out.reshape(B, 1)


# speedup vs baseline: 1.0426x; 1.0069x over previous
import functools

import jax
import jax.numpy as jnp
from jax import lax
from jax.experimental import pallas as pl
from jax.experimental.pallas import tpu as pltpu
from jax.experimental.pallas import tpu_sc as plsc

B = 16384
N_SPARSE = 26
N_DENSE = 7
N_COLS = N_SPARSE + N_DENSE
VOCAB = 1000

NUM_CORES = 2
NUM_SUBCORES = 16
NW = NUM_CORES * NUM_SUBCORES
ROWS_PER_W = B // NW
LANES = 16
GROUPS = ROWS_PER_W // LANES
TABLE_WORDS = N_SPARSE * VOCAB
FBLOCKS = (3, 6, 8, 9)


@functools.partial(
    pl.kernel,
    mesh=plsc.VectorSubcoreMesh(core_axis_name="c", subcore_axis_name="s"),
    out_type=jax.ShapeDtypeStruct((B,), jnp.float32),
    compiler_params=pltpu.CompilerParams(needs_layout_passes=False),
    scratch_types=[
        pltpu.VMEM((N_COLS, ROWS_PER_W), jnp.float32),
        pltpu.VMEM((TABLE_WORDS + 16,), jnp.float32),
        pltpu.VMEM((ROWS_PER_W,), jnp.float32),
        pltpu.SemaphoreType.DMA,
        pltpu.SemaphoreType.DMA,
        pltpu.SemaphoreType.DMA,
        pltpu.SemaphoreType.DMA,
        pltpu.SemaphoreType.DMA,
    ],
)
def _linear_logit_sc(
    xt_hbm, t_hbm, out_hbm, xv, tv, ov, semxd, st0, st1, st2, st3
):
    wid = lax.axis_index("s") * NUM_CORES + lax.axis_index("c")
    base = wid * ROWS_PER_W
    xcp = pltpu.async_copy(
        xt_hbm.at[:, pl.ds(base, ROWS_PER_W)], xv, semxd
    )
    pltpu.sync_copy(
        t_hbm.at[pl.ds(TABLE_WORDS, 8)], tv.at[pl.ds(TABLE_WORDS, 8)]
    )
    tsems = [st0, st1, st2, st3]
    tcps = []
    f0 = 0
    fstarts = []
    for bi, nf in enumerate(FBLOCKS):
        fstarts.append(f0)
        tcps.append(
            pltpu.async_copy(
                t_hbm.at[pl.ds(f0 * VOCAB, nf * VOCAB)],
                tv.at[pl.ds(f0 * VOCAB, nf * VOCAB)],
                tsems[bi],
            )
        )
        f0 += nf

    wvec = tv[pl.ds(TABLE_WORDS, LANES)]
    wsplat = [wvec[d] for d in range(N_DENSE)]
    xcp.wait()

    @plsc.parallel_loop(0, GROUPS)
    def dense(g):
        r0 = g * LANES
        acc = xv[N_SPARSE, pl.ds(r0, LANES)] * wsplat[0]
        for d in range(1, N_DENSE):
            acc = acc + xv[N_SPARSE + d, pl.ds(r0, LANES)] * wsplat[d]
        ov[pl.ds(r0, LANES)] = acc

    for bi, nf in enumerate(FBLOCKS):
        tcps[bi].wait()
        fs = fstarts[bi]

        @plsc.parallel_loop(0, GROUPS)
        def sparse(g, fs=fs, nf=nf):
            r0 = g * LANES
            acc = ov[pl.ds(r0, LANES)]
            for f in range(fs, fs + nf):
                ids = xv[f, pl.ds(r0, LANES)].astype(jnp.int32)
                acc = acc + plsc.load_gather(tv, [ids + f * VOCAB])
            ov[pl.ds(r0, LANES)] = acc

    pltpu.sync_copy(ov, out_hbm.at[pl.ds(base, ROWS_PER_W)])


def kernel(X, emb_tables, dense_weight):
    xt = X.T
    t_flat = jnp.concatenate([
        emb_tables.reshape(-1),
        jnp.pad(dense_weight.reshape(-1), (0, 8 - N_DENSE)),
    ])
    out = _linear_logit_sc(xt, t_flat)
    return out.reshape(B, 1)
